# Initial kernel scaffold; baseline (speedup 1.0000x reference)
import jax
import jax.numpy as jnp
from jax.experimental import pallas as pl

_B = 16384
_C = 1000
_K = 5
_BLK = 512


def _loss_block_kernel(x_ref, y_ref, o_ref):
    i = pl.program_id(0)
    x = x_ref[...]
    y = y_ref[...]
    col = jax.lax.broadcasted_iota(jnp.int32, (_BLK, _C), 1)
    z = jnp.zeros((_BLK, _C), jnp.float32)
    for k in range(_K):
        yk = y[:, k:k + 1]
        z = jnp.where(col == yk, 1.0, z)
    s = jnp.sum(jnp.maximum(x, 0.0) - x * z + jnp.log1p(jnp.exp(-jnp.abs(x))))

    @pl.when(i == 0)
    def _init():
        o_ref[0, 0] = 0.0

    o_ref[0, 0] += s


def kernel(out, y_inds):
    y32 = y_inds.astype(jnp.int32)
    total = pl.pallas_call(
        _loss_block_kernel,
        grid=(_B // _BLK,),
        in_specs=[
            pl.BlockSpec((_BLK, _C), lambda i: (i, 0)),
            pl.BlockSpec((_BLK, _K), lambda i: (i, 0)),
        ],
        out_specs=pl.BlockSpec((1, 1), lambda i: (0, 0)),
        out_shape=jax.ShapeDtypeStruct((1, 1), jnp.float32),
    )(out, y32)
    return (total[0, 0] / (_B * _C)).astype(out.dtype)


# speedup vs baseline: 3.9857x; 3.9857x over previous
import jax
import jax.numpy as jnp
from jax.experimental import pallas as pl

_B = 16384
_C = 1000
_K = 5
_BLK = 512


def _loss_block_kernel(x_ref, y_ref, o_ref):
    i = pl.program_id(0)
    x = x_ref[...]
    y = y_ref[...]
    col = jax.lax.broadcasted_iota(jnp.int32, (_BLK, _C), 1)
    z = jnp.zeros((_BLK, _C), jnp.float32)
    for k in range(_K):
        yk = y[:, k:k + 1]
        z = jnp.where(col == yk, 1.0, z)
    s = jnp.sum(jnp.maximum(x, 0.0) - x * z + jnp.log1p(jnp.exp(-jnp.abs(x))))

    @pl.when(i == 0)
    def _init():
        o_ref[...] = jnp.zeros((1, 1), jnp.float32)

    o_ref[...] += s.reshape(1, 1)


def kernel(out, y_inds):
    y32 = y_inds.astype(jnp.int32)
    total = pl.pallas_call(
        _loss_block_kernel,
        grid=(_B // _BLK,),
        in_specs=[
            pl.BlockSpec((_BLK, _C), lambda i: (i, 0)),
            pl.BlockSpec((_BLK, _K), lambda i: (i, 0)),
        ],
        out_specs=pl.BlockSpec((1, 1), lambda i: (0, 0)),
        out_shape=jax.ShapeDtypeStruct((1, 1), jnp.float32),
    )(out, y32)
    return (total[0, 0] / (_B * _C)).astype(out.dtype)
